# Initial kernel scaffold; baseline (speedup 1.0000x reference)
import jax
import jax.numpy as jnp
from jax.experimental import pallas as pl


def kernel(x, edge_index, edge_attr, batch, params):
    raise NotImplementedError("write your pallas kernel here")


# speedup vs baseline: 1.2848x; 1.2848x over previous
import functools
import jax
import jax.numpy as jnp
from jax.experimental import pallas as pl

G = 16
K = 3
RATIO = 0.11


def _seg_sum(data, ids, num):
    return jax.ops.segment_sum(data, ids, num_segments=num)


def _S(u, src, dst, ew, n):
    g = u[src]
    if ew is not None:
        g = g * ew[:, None]
    return _seg_sum(g, dst, n)


def _cheb_layer(x, src, dst, ew, dis, W, b, n):
    xd = dis[:, None] * x
    t1 = -dis[:, None] * _S(xd, src, dst, ew, n)
    c = t1 @ W[2]
    out = x @ (W[0] - W[2]) + t1 @ W[1]
    out = out - 2.0 * dis[:, None] * _S(dis[:, None] * c, src, dst, ew, n)
    return out + b


def _gcm(h, src, dst, nmask, cntm, Wrel, brel, Wroot, n):
    y = h @ Wrel
    s = nmask[:, None] * _S(y, src, dst, None, n)
    return s / cntm[:, None] + brel + h @ Wroot


def _topk_sel(score_r, batch, nmask, n):
    n_act = _seg_sum(nmask, batch, G)
    k_g = jnp.ceil(RATIO * n_act)
    key_ = batch.astype(jnp.float32) * 10.0 - score_r
    order = jnp.argsort(key_)
    counts = jnp.bincount(batch, length=G)
    starts = jnp.concatenate([jnp.zeros((1,), counts.dtype), jnp.cumsum(counts)[:-1]])
    pos = jnp.arange(n)
    rank = jnp.zeros((n,), jnp.int32).at[order].set(
        (pos - starts[batch[order]]).astype(jnp.int32))
    sel = jnp.logical_and(rank.astype(jnp.float32) < k_g[batch], nmask > 0)
    return sel.astype(jnp.float32)


def _combine_kernel(a_ref, b_ref, o_ref):
    a = a_ref[...]
    b = b_ref[...]
    sa = jnp.exp(a - jnp.max(a, axis=1, keepdims=True))
    sa = sa / jnp.sum(sa, axis=1, keepdims=True)
    sb = jnp.exp(b - jnp.max(b, axis=1, keepdims=True))
    sb = sb / jnp.sum(sb, axis=1, keepdims=True)
    o_ref[...] = sa + sb


def kernel(x, edge_index, edge_attr, batch, params):
    p = params
    n = x.shape[0]
    src, dst = edge_index[0], edge_index[1]
    ew = edge_attr

    onehot = (batch[None, :] == jnp.arange(G, dtype=batch.dtype)[:, None]
              ).astype(jnp.float32)

    deg = _seg_sum(ew, src, n)
    dis = jnp.where(deg > 0, jax.lax.rsqrt(jnp.maximum(deg, 1e-12)), 0.0)

    h = jax.nn.relu(_cheb_layer(x, src, dst, ew, dis, p["a_c1_W"], p["a_c1_b"], n))
    h = jax.nn.relu(_cheb_layer(h, src, dst, ew, dis, p["a_c2_W"], p["a_c2_b"], n))
    h = jax.nn.relu(_cheb_layer(h, src, dst, ew, dis, p["a_c3_W"], p["a_c3_b"], n))
    cnt_a = onehot @ jnp.ones((n, 1), jnp.float32)
    pooled = (onehot @ h) / jnp.maximum(cnt_a, 1.0)
    x1 = pooled @ p["a_l_W"] + p["a_l_b"]

    nmask = jnp.ones((n,), jnp.float32)
    cnt = nmask * _S(nmask[:, None], src, dst, None, n)[:, 0]
    cntm = jnp.maximum(cnt, 1.0)

    def gmp(hh, nm):
        s = onehot @ hh
        c = onehot @ nm[:, None]
        return s / jnp.maximum(c, 1.0)

    h = jax.nn.relu(_gcm(x, src, dst, nmask, cntm, p["b_c1_Wrel"], p["b_c1_brel"],
                         p["b_c1_Wroot"], n)) * nmask[:, None]
    xs = [gmp(h, nmask)]
    for i in range(5):
        h = jax.nn.relu(_gcm(h, src, dst, nmask, cntm, p["b_cs%d_Wrel" % i],
                             p["b_cs%d_brel" % i], p["b_cs%d_Wroot" % i], n))
        h = h * nmask[:, None]
        xs.append(gmp(h, nmask))
        if i % 2 == 0 and i < 4:
            j = i // 2
            score = jnp.tanh(_gcm(h, src, dst, nmask, cntm, p["b_p%d_Wrel" % j],
                                  p["b_p%d_brel" % j], p["b_p%d_Wroot" % j], n)[:, 0])
            score_r = jnp.where(nmask > 0, score, -2.0)
            sel = _topk_sel(score_r, batch, nmask, n)
            h = h * score[:, None] * sel[:, None]
            nmask = sel
            cnt = nmask * _S(nmask[:, None], src, dst, None, n)[:, 0]
            cntm = jnp.maximum(cnt, 1.0)

    hcat = jnp.concatenate(xs, axis=1)
    z = jax.nn.relu(hcat @ p["b_l1_W"] + p["b_l1_b"])
    x2 = z @ p["b_l2_W"] + p["b_l2_b"]

    out = pl.pallas_call(
        _combine_kernel,
        out_shape=jax.ShapeDtypeStruct((G, 2), jnp.float32),
    )(x1, x2)
    return out


# speedup vs baseline: 5.2407x; 4.0790x over previous
import functools
import jax
import jax.numpy as jnp
from jax import lax
from jax.experimental import pallas as pl
from jax.experimental.pallas import tpu as pltpu
from jax.experimental.pallas import tpu_sc as plsc

G = 16
RATIO = 0.11

NC = 2
NS = 16
NW = NC * NS
CHUNK = 80
RPW = 640
NROWS = NS * RPW


def _feat(F):
    return (F,) if F > 1 else ()


@functools.lru_cache(None)
def _make_seg(E, F, weighted, edge_mode):
    feat = _feat(F)
    epw = E // NW
    nchunks = epw // CHUNK
    assert E % NW == 0 and epw % CHUNK == 0

    def body(*refs):
        if edge_mode:
            (u_hbm, dst_hbm, out_hbm, dst_v, rows_v, acc) = refs
        elif weighted:
            (u_hbm, src_hbm, dst_hbm, ew_hbm, out_hbm,
             src_v, dst_v, ew_v, rows_v, acc) = refs
        else:
            (u_hbm, src_hbm, dst_hbm, out_hbm,
             src_v, dst_v, rows_v, acc) = refs

        cid = lax.axis_index("c")
        sid = lax.axis_index("s")
        wid = cid * NS + sid

        zero16 = jnp.zeros((16,), jnp.float32)

        def zbody(r, c):
            if F == 1:
                rows_v[pl.ds(r * 16, 16)] = zero16
            else:
                for f in range(F // 16):
                    rows_v[r, pl.ds(f * 16, 16)] = zero16
            return c

        lax.fori_loop(0, CHUNK // 16 if F == 1 else CHUNK, zbody, 0)
        rbase = sid * RPW
        for b in range(RPW // CHUNK):
            pltpu.sync_copy(rows_v, acc.at[pl.ds(rbase + b * CHUNK, CHUNK)])
        plsc.subcore_barrier()

        ebase = wid * epw

        def cbody(i, c):
            base = ebase + i * CHUNK
            pltpu.sync_copy(dst_hbm.at[pl.ds(base, CHUNK)], dst_v)
            if edge_mode:
                pltpu.sync_copy(u_hbm.at[pl.ds(base, CHUNK)], rows_v)
            else:
                pltpu.sync_copy(src_hbm.at[pl.ds(base, CHUNK)], src_v)
                pltpu.sync_copy(u_hbm.at[src_v], rows_v)
            if weighted:
                pltpu.sync_copy(ew_hbm.at[pl.ds(base, CHUNK)], ew_v)

                def wbody(g, cc):
                    wvec = ew_v[pl.ds(g * 16, 16)]
                    for jj in range(16):
                        w = wvec[jj]
                        e = g * 16 + jj
                        for f in range(F // 16):
                            sl = pl.ds(f * 16, 16)
                            rows_v[e, sl] = rows_v[e, sl] * w
                    return cc

                lax.fori_loop(0, CHUNK // 16, wbody, 0)
            pltpu.sync_copy(rows_v, acc.at[dst_v], add=True)
            return c

        lax.fori_loop(0, nchunks, cbody, 0)
        plsc.subcore_barrier()
        pltpu.sync_copy(acc.at[pl.ds(rbase, RPW)],
                        out_hbm.at[cid, pl.ds(rbase, RPW)])

    scratch = []
    if not edge_mode:
        scratch.append(pltpu.VMEM((CHUNK,), jnp.int32))
    scratch.append(pltpu.VMEM((CHUNK,), jnp.int32))
    if weighted:
        scratch.append(pltpu.VMEM((CHUNK,), jnp.float32))
    scratch.append(pltpu.VMEM((CHUNK,) + feat, jnp.float32))
    scratch.append(pltpu.VMEM_SHARED((NROWS,) + feat, jnp.float32))

    return pl.kernel(
        body,
        out_type=jax.ShapeDtypeStruct((NC, NROWS) + feat, jnp.float32),
        mesh=plsc.VectorSubcoreMesh(core_axis_name="c", subcore_axis_name="s"),
        scratch_types=scratch,
        compiler_params=pltpu.CompilerParams(use_tc_tiling_on_sc=False),
    )


def _S(u, src, dst, ew, n):
    k = _make_seg(src.shape[0], u.shape[1], ew is not None, False)
    args = (u, src, dst) + ((ew,) if ew is not None else ())
    p = k(*args)
    return (p[0] + p[1])[:n]


def _S1(u, src, dst, n):
    k = _make_seg(src.shape[0], 1, False, False)
    p = k(u, src, dst)
    return (p[0] + p[1])[:n]


def _Sedge(data, ids, n):
    k = _make_seg(ids.shape[0], 1, False, True)
    p = k(data, ids)
    return (p[0] + p[1])[:n]


def _cheb_layer(x, src, dst, ew, dis, W, b, n):
    xd = dis[:, None] * x
    t1 = -dis[:, None] * _S(xd, src, dst, ew, n)
    c = t1 @ W[2]
    out = x @ (W[0] - W[2]) + t1 @ W[1]
    out = out - 2.0 * dis[:, None] * _S(dis[:, None] * c, src, dst, ew, n)
    return out + b


def _gcm(h, src, dst, nmask, cntm, Wrel, brel, Wroot, n):
    y = h @ Wrel
    if Wrel.shape[1] == 1:
        s = nmask * _S1(y[:, 0], src, dst, n)
        return s / cntm + brel[0] + (h @ Wroot)[:, 0]
    s = nmask[:, None] * _S(y, src, dst, None, n)
    return s / cntm[:, None] + brel + h @ Wroot


def _topk_sel(score_r, batch, nmask, n):
    n_act = jax.ops.segment_sum(nmask, batch, num_segments=G)
    k_g = jnp.ceil(RATIO * n_act)
    key_ = batch.astype(jnp.float32) * 10.0 - score_r
    order = jnp.argsort(key_)
    counts = jnp.bincount(batch, length=G)
    starts = jnp.concatenate([jnp.zeros((1,), counts.dtype), jnp.cumsum(counts)[:-1]])
    pos = jnp.arange(n)
    rank = jnp.zeros((n,), jnp.int32).at[order].set(
        (pos - starts[batch[order]]).astype(jnp.int32))
    sel = jnp.logical_and(rank.astype(jnp.float32) < k_g[batch], nmask > 0)
    return sel.astype(jnp.float32)


def _combine_kernel(a_ref, b_ref, o_ref):
    a = a_ref[...]
    b = b_ref[...]
    sa = jnp.exp(a - jnp.max(a, axis=1, keepdims=True))
    sa = sa / jnp.sum(sa, axis=1, keepdims=True)
    sb = jnp.exp(b - jnp.max(b, axis=1, keepdims=True))
    sb = sb / jnp.sum(sb, axis=1, keepdims=True)
    o_ref[...] = sa + sb


def kernel(x, edge_index, edge_attr, batch, params):
    p = params
    n = x.shape[0]
    src, dst = edge_index[0], edge_index[1]
    ew = edge_attr

    onehot = (batch[None, :] == jnp.arange(G, dtype=batch.dtype)[:, None]
              ).astype(jnp.float32)

    deg = _Sedge(ew, src, n)
    dis = jnp.where(deg > 0, jax.lax.rsqrt(jnp.maximum(deg, 1e-12)), 0.0)

    h = jax.nn.relu(_cheb_layer(x, src, dst, ew, dis, p["a_c1_W"], p["a_c1_b"], n))
    h = jax.nn.relu(_cheb_layer(h, src, dst, ew, dis, p["a_c2_W"], p["a_c2_b"], n))
    h = jax.nn.relu(_cheb_layer(h, src, dst, ew, dis, p["a_c3_W"], p["a_c3_b"], n))
    cnt_a = onehot @ jnp.ones((n, 1), jnp.float32)
    pooled = (onehot @ h) / jnp.maximum(cnt_a, 1.0)
    x1 = pooled @ p["a_l_W"] + p["a_l_b"]

    nmask = jnp.ones((n,), jnp.float32)
    cnt = nmask * _S1(nmask, src, dst, n)
    cntm = jnp.maximum(cnt, 1.0)

    def gmp(hh, nm):
        s = onehot @ hh
        c = onehot @ nm[:, None]
        return s / jnp.maximum(c, 1.0)

    h = jax.nn.relu(_gcm(x, src, dst, nmask, cntm, p["b_c1_Wrel"], p["b_c1_brel"],
                         p["b_c1_Wroot"], n)) * nmask[:, None]
    xs = [gmp(h, nmask)]
    for i in range(5):
        h = jax.nn.relu(_gcm(h, src, dst, nmask, cntm, p["b_cs%d_Wrel" % i],
                             p["b_cs%d_brel" % i], p["b_cs%d_Wroot" % i], n))
        h = h * nmask[:, None]
        xs.append(gmp(h, nmask))
        if i % 2 == 0 and i < 4:
            j = i // 2
            score = jnp.tanh(_gcm(h, src, dst, nmask, cntm, p["b_p%d_Wrel" % j],
                                  p["b_p%d_brel" % j], p["b_p%d_Wroot" % j], n))
            score_r = jnp.where(nmask > 0, score, -2.0)
            sel = _topk_sel(score_r, batch, nmask, n)
            h = h * score[:, None] * sel[:, None]
            nmask = sel
            cnt = nmask * _S1(nmask, src, dst, n)
            cntm = jnp.maximum(cnt, 1.0)

    hcat = jnp.concatenate(xs, axis=1)
    z = jax.nn.relu(hcat @ p["b_l1_W"] + p["b_l1_b"])
    x2 = z @ p["b_l2_W"] + p["b_l2_b"]

    out = pl.pallas_call(
        _combine_kernel,
        out_shape=jax.ShapeDtypeStruct((G, 2), jnp.float32),
    )(x1, x2)
    return out


# speedup vs baseline: 8.6513x; 1.6508x over previous
import functools
import jax
import jax.numpy as jnp
from jax import lax
from jax.experimental import pallas as pl
from jax.experimental.pallas import tpu as pltpu
from jax.experimental.pallas import tpu_sc as plsc

G = 16
RATIO = 0.11

NC = 2
NS = 16
NW = NC * NS
CHUNK = 128
RPW = 640
NROWS = NS * RPW


def _feat(F):
    return (F,) if F > 1 else ()


@functools.lru_cache(None)
def _make_seg(nch, F, weighted, edge_mode):
    feat = _feat(F)

    def body(*refs):
        if edge_mode:
            (u_hbm, dst_hbm, out_hbm, dst_a, rows0, rows1, acc,
             sem0, sem1) = refs
        elif weighted:
            (u_hbm, src_hbm, dst_hbm, ew_hbm, out_hbm,
             src_a, dst_a, ew_a, rows0, rows1, acc, sem0, sem1) = refs
        else:
            (u_hbm, src_hbm, dst_hbm, out_hbm,
             src_a, dst_a, rows0, rows1, acc, sem0, sem1) = refs

        cid = lax.axis_index("c")
        sid = lax.axis_index("s")
        wid = cid * NS + sid
        cbase = wid * nch

        pltpu.sync_copy(dst_hbm.at[pl.ds(cbase, nch)], dst_a)
        if not edge_mode:
            pltpu.sync_copy(src_hbm.at[pl.ds(cbase, nch)], src_a)
        if weighted:
            pltpu.sync_copy(ew_hbm.at[pl.ds(cbase, nch)], ew_a)

        zero16 = jnp.zeros((16,), jnp.float32)

        def zbody(r, c):
            if F == 1:
                rows0[pl.ds(r * 16, 16)] = zero16
            else:
                for f in range(F // 16):
                    rows0[r, pl.ds(f * 16, 16)] = zero16
            return c

        lax.fori_loop(0, CHUNK // 16 if F == 1 else CHUNK, zbody, 0)
        rbase = sid * RPW
        for b in range(RPW // CHUNK):
            pltpu.sync_copy(rows0, acc.at[pl.ds(rbase + b * CHUNK, CHUNK)])
        plsc.subcore_barrier()

        def start(i, buf, sem):
            if edge_mode:
                pltpu.async_copy(u_hbm.at[cbase + i], buf, sem)
            else:
                pltpu.async_copy(u_hbm.at[src_a.at[i]], buf, sem)

        def wait(i, buf, sem):
            if edge_mode:
                pltpu.make_async_copy(u_hbm.at[cbase + i], buf, sem).wait()
            else:
                pltpu.make_async_copy(u_hbm.at[src_a.at[i]], buf, sem).wait()

        def mult(i, buf):
            if not weighted:
                return

            def wbody(g, cc):
                wvec = ew_a[i, pl.ds(g * 16, 16)]
                for jj in range(16):
                    w = wvec[jj]
                    e = g * 16 + jj
                    for f in range(F // 16):
                        sl = pl.ds(f * 16, 16)
                        buf[e, sl] = buf[e, sl] * w
                return cc

            lax.fori_loop(0, CHUNK // 16, wbody, 0)

        def scatter(i, buf):
            pltpu.sync_copy(buf, acc.at[dst_a.at[i]], add=True)

        start(0, rows0, sem0)

        def pair(ip, c):
            i0 = ip * 2
            start(i0 + 1, rows1, sem1)
            wait(i0, rows0, sem0)
            mult(i0, rows0)
            scatter(i0, rows0)
            start(i0 + 2, rows0, sem0)
            wait(i0 + 1, rows1, sem1)
            mult(i0 + 1, rows1)
            scatter(i0 + 1, rows1)
            return c

        lax.fori_loop(0, (nch - 1) // 2, pair, 0)
        last = nch - 1
        wait(last, rows0, sem0)
        mult(last, rows0)
        scatter(last, rows0)

        plsc.subcore_barrier()
        pltpu.sync_copy(acc.at[pl.ds(rbase, RPW)],
                        out_hbm.at[cid, pl.ds(rbase, RPW)])

    scratch = []
    if not edge_mode:
        scratch.append(pltpu.VMEM((nch, CHUNK), jnp.int32))
    scratch.append(pltpu.VMEM((nch, CHUNK), jnp.int32))
    if weighted:
        scratch.append(pltpu.VMEM((nch, CHUNK), jnp.float32))
    scratch.append(pltpu.VMEM((CHUNK,) + feat, jnp.float32))
    scratch.append(pltpu.VMEM((CHUNK,) + feat, jnp.float32))
    scratch.append(pltpu.VMEM_SHARED((NROWS,) + feat, jnp.float32))
    scratch.append(pltpu.SemaphoreType.DMA)
    scratch.append(pltpu.SemaphoreType.DMA)

    return pl.kernel(
        body,
        out_type=jax.ShapeDtypeStruct((NC, NROWS) + feat, jnp.float32),
        mesh=plsc.VectorSubcoreMesh(core_axis_name="c", subcore_axis_name="s"),
        scratch_types=scratch,
        compiler_params=pltpu.CompilerParams(use_tc_tiling_on_sc=False),
    )


def _pad_edges(src, dst, ew):
    E = src.shape[0]
    nch = -(-E // (NW * CHUNK))
    pad = NW * nch * CHUNK - E
    srcp = jnp.concatenate([src, jnp.zeros((pad,), src.dtype)])
    dstp = jnp.concatenate([dst, jnp.full((pad,), NROWS - 1, dst.dtype)])
    ewp = jnp.concatenate([ew, jnp.zeros((pad,), ew.dtype)])
    shape = (NW * nch, CHUNK)
    return (srcp.reshape(shape), dstp.reshape(shape), ewp.reshape(shape), nch)


MAX_SLAB = 80


def _S(u, srcp, dstp, ewp, nch, n):
    F = u.shape[1]
    outs = []
    a = 0
    while a < F:
        w = min(MAX_SLAB, F - a)
        k = _make_seg(nch, w, ewp is not None, False)
        args = (u[:, a:a + w], srcp, dstp) + ((ewp,) if ewp is not None else ())
        p = k(*args)
        outs.append((p[0] + p[1])[:n])
        a += w
    return outs[0] if len(outs) == 1 else jnp.concatenate(outs, axis=1)


def _S1(u, srcp, dstp, nch, n):
    k = _make_seg(nch, 1, False, False)
    p = k(u, srcp, dstp)
    return (p[0] + p[1])[:n]


def _Sedge(datap, idsp, nch, n):
    k = _make_seg(nch, 1, False, True)
    p = k(datap, idsp)
    return (p[0] + p[1])[:n]


def _cheb_layer(x, srcp, dstp, ewp, nch, dis, W, b, n):
    xd = dis[:, None] * x
    t1 = -dis[:, None] * _S(xd, srcp, dstp, ewp, nch, n)
    c = t1 @ W[2]
    out = x @ (W[0] - W[2]) + t1 @ W[1]
    out = out - 2.0 * dis[:, None] * _S(dis[:, None] * c, srcp, dstp, ewp, nch, n)
    return out + b


def _gcm(h, srcp, dstp, nch, nmask, cntm, Wrel, brel, Wroot, n):
    y = h @ Wrel
    if Wrel.shape[1] == 1:
        s = nmask * _S1(y[:, 0], srcp, dstp, nch, n)
        return s / cntm + brel[0] + (h @ Wroot)[:, 0]
    s = nmask[:, None] * _S(y, srcp, dstp, None, nch, n)
    return s / cntm[:, None] + brel + h @ Wroot


def _topk_sel(score_r, batch, nmask, n):
    n_act = jax.ops.segment_sum(nmask, batch, num_segments=G)
    k_g = jnp.ceil(RATIO * n_act)
    key_ = batch.astype(jnp.float32) * 10.0 - score_r
    order = jnp.argsort(key_)
    counts = jnp.bincount(batch, length=G)
    starts = jnp.concatenate([jnp.zeros((1,), counts.dtype), jnp.cumsum(counts)[:-1]])
    pos = jnp.arange(n)
    rank = jnp.zeros((n,), jnp.int32).at[order].set(
        (pos - starts[batch[order]]).astype(jnp.int32))
    sel = jnp.logical_and(rank.astype(jnp.float32) < k_g[batch], nmask > 0)
    return sel.astype(jnp.float32)


def _combine_kernel(a_ref, b_ref, o_ref):
    a = a_ref[...]
    b = b_ref[...]
    sa = jnp.exp(a - jnp.max(a, axis=1, keepdims=True))
    sa = sa / jnp.sum(sa, axis=1, keepdims=True)
    sb = jnp.exp(b - jnp.max(b, axis=1, keepdims=True))
    sb = sb / jnp.sum(sb, axis=1, keepdims=True)
    o_ref[...] = sa + sb


def kernel(x, edge_index, edge_attr, batch, params):
    p = params
    n = x.shape[0]
    src, dst = edge_index[0], edge_index[1]
    ew = edge_attr
    srcp, dstp, ewp, nch = _pad_edges(src, dst, ew)

    onehot = (batch[None, :] == jnp.arange(G, dtype=batch.dtype)[:, None]
              ).astype(jnp.float32)

    deg = _Sedge(ewp, srcp, nch, n)
    dis = jnp.where(deg > 0, jax.lax.rsqrt(jnp.maximum(deg, 1e-12)), 0.0)

    h = jax.nn.relu(_cheb_layer(x, srcp, dstp, ewp, nch, dis, p["a_c1_W"], p["a_c1_b"], n))
    h = jax.nn.relu(_cheb_layer(h, srcp, dstp, ewp, nch, dis, p["a_c2_W"], p["a_c2_b"], n))
    h = jax.nn.relu(_cheb_layer(h, srcp, dstp, ewp, nch, dis, p["a_c3_W"], p["a_c3_b"], n))
    cnt_a = onehot @ jnp.ones((n, 1), jnp.float32)
    pooled = (onehot @ h) / jnp.maximum(cnt_a, 1.0)
    x1 = pooled @ p["a_l_W"] + p["a_l_b"]

    nmask = jnp.ones((n,), jnp.float32)
    cnt = nmask * _S1(nmask, srcp, dstp, nch, n)
    cntm = jnp.maximum(cnt, 1.0)

    def gmp(hh, nm):
        s = onehot @ hh
        c = onehot @ nm[:, None]
        return s / jnp.maximum(c, 1.0)

    h = jax.nn.relu(_gcm(x, srcp, dstp, nch, nmask, cntm, p["b_c1_Wrel"], p["b_c1_brel"],
                         p["b_c1_Wroot"], n)) * nmask[:, None]
    xs = [gmp(h, nmask)]
    for i in range(5):
        h = jax.nn.relu(_gcm(h, srcp, dstp, nch, nmask, cntm, p["b_cs%d_Wrel" % i],
                             p["b_cs%d_brel" % i], p["b_cs%d_Wroot" % i], n))
        h = h * nmask[:, None]
        xs.append(gmp(h, nmask))
        if i % 2 == 0 and i < 4:
            j = i // 2
            score = jnp.tanh(_gcm(h, srcp, dstp, nch, nmask, cntm, p["b_p%d_Wrel" % j],
                                  p["b_p%d_brel" % j], p["b_p%d_Wroot" % j], n))
            score_r = jnp.where(nmask > 0, score, -2.0)
            sel = _topk_sel(score_r, batch, nmask, n)
            h = h * score[:, None] * sel[:, None]
            nmask = sel
            cnt = nmask * _S1(nmask, srcp, dstp, nch, n)
            cntm = jnp.maximum(cnt, 1.0)

    hcat = jnp.concatenate(xs, axis=1)
    z = jax.nn.relu(hcat @ p["b_l1_W"] + p["b_l1_b"])
    x2 = z @ p["b_l2_W"] + p["b_l2_b"]

    out = pl.pallas_call(
        _combine_kernel,
        out_shape=jax.ShapeDtypeStruct((G, 2), jnp.float32),
    )(x1, x2)
    return out
